# baseline (speedup 1.0000x reference)
import functools

import jax
import jax.numpy as jnp
from jax import lax
from jax.experimental import pallas as pl
from jax.experimental.pallas import tpu as pltpu
from jax.experimental.pallas import tpu_sc as plsc

N_TOK = 4096
D = 768
E = 8
DH = 256
GLANES = 128
TILE = 512
FTILE = 256
NDISP = 2 * N_TOK + E * FTILE
NFT = NDISP // FTILE
NW = 32
TPW = N_TOK // NW
CHUNK = 64
NEG = -1e30


def _dg(a, b):
    return lax.dot_general(a, b, (((1,), (1,)), ((), ())),
                           preferred_element_type=jnp.float32)


def _top2(logits):
    iota = lax.broadcasted_iota(jnp.int32, logits.shape, 1)
    m1 = jnp.max(logits, axis=1, keepdims=True)
    i1 = jnp.min(jnp.where(logits == m1, iota, GLANES), axis=1, keepdims=True)
    lm = jnp.where(iota == i1, NEG, logits)
    m2 = jnp.max(lm, axis=1, keepdims=True)
    i2 = jnp.min(jnp.where(lm == m2, iota, GLANES), axis=1, keepdims=True)
    w1 = 1.0 / (1.0 + jnp.exp(m2 - m1))
    w2 = 1.0 - w1
    return i1, i2, w1, w2


def _a_body(x_ref, Wi_ref, bi_ref, Wv_ref, bv_ref, Wo_ref, bo_ref, Wg_ref,
            bg_ref, x2_ref, pos0_ref, pos1_ref, w0_ref, w1_ref, te_ref,
            carry_ref, meta_ref):
    i = pl.program_id(0)
    f32 = jnp.float32

    @pl.when(i == 0)
    def _init():
        carry_ref[...] = jnp.zeros((1, 128), f32)

    @pl.when(i < N_TOK // TILE)
    def _proj():
        x1 = _dg(x_ref[...], Wi_ref[...]) + bi_ref[...]
        v = _dg(x1, Wv_ref[...]) + bv_ref[...]
        x2 = _dg(v, Wo_ref[...]) + bo_ref[...]
        logits = _dg(x2, Wg_ref[...]) + bg_ref[...]
        i1, i2, wa, wb = _top2(logits)
        x2_ref[...] = x2
        w0_ref[...] = jnp.broadcast_to(wa, (TILE, 16))
        w1_ref[...] = jnp.broadcast_to(wb, (TILE, 16))
        ioE = lax.broadcasted_iota(jnp.int32, (TILE, E), 1)
        m = (ioE == i1).astype(f32) + (ioE == i2).astype(f32)
        r = lax.broadcasted_iota(jnp.int32, (TILE, TILE), 0)
        c = lax.broadcasted_iota(jnp.int32, (TILE, TILE), 1)
        tri = (c < r).astype(f32)
        pref = lax.dot_general(tri, m, (((1,), (0,)), ((), ())),
                               preferred_element_type=f32)
        mp = pref + carry_ref[0:1, 0:E]
        meta_ref[pl.ds(i * TILE, TILE), 0:E] = mp
        meta_ref[pl.ds(i * TILE, TILE), E:E + 1] = i1.astype(f32)
        meta_ref[pl.ds(i * TILE, TILE), E + 1:E + 2] = i2.astype(f32)
        carry_ref[0:1, 0:E] = carry_ref[0:1, 0:E] + jnp.sum(m, axis=0,
                                                            keepdims=True)

    @pl.when(i == N_TOK // TILE)
    def _plan():
        counts = carry_ref[0:1, 0:E].astype(jnp.int32)
        pc = ((counts + (FTILE - 1)) // FTILE) * FTILE
        er = lax.broadcasted_iota(jnp.int32, (E, E), 0)
        ec = lax.broadcasted_iota(jnp.int32, (E, E), 1)
        ut = (er < ec).astype(f32)
        seg = lax.dot_general(pc.astype(f32), ut, (((1,), (0,)), ((), ())),
                              preferred_element_type=f32)
        cumt = (seg.astype(jnp.int32) + pc) // FTILE
        tio = lax.broadcasted_iota(jnp.int32, (E, 128), 1)
        te = jnp.sum((tio >= cumt.reshape(E, 1)).astype(jnp.int32), axis=0,
                     keepdims=True)
        te_ref[...] = jnp.minimum(te, E - 1)
        for tt in range(N_TOK // TILE):
            blk = meta_ref[pl.ds(tt * TILE, TILE), :]
            mp = blk[:, 0:E] + seg
            i1f = blk[:, E:E + 1]
            i2f = blk[:, E + 1:E + 2]
            ioEf = lax.broadcasted_iota(jnp.int32, (TILE, E), 1).astype(f32)
            p0 = jnp.sum(jnp.where(ioEf == i1f, mp, 0.0), axis=1, keepdims=True)
            p1 = jnp.sum(jnp.where(ioEf == i2f, mp, 0.0), axis=1, keepdims=True)
            rpt = TILE // 128
            pos0_ref[pl.ds(tt * rpt, rpt), :, :] = p0.reshape(rpt, 2, 64).astype(jnp.int32)
            pos1_ref[pl.ds(tt * rpt, rpt), :, :] = p1.reshape(rpt, 2, 64).astype(jnp.int32)


def _route(x, Wi, bi, Wv, bv, Wo, bo, Wg, bg):
    full2 = lambda i: (0, 0)
    xmap = lambda i: (jnp.minimum(i, N_TOK // TILE - 1), 0)
    rpt = TILE // 128
    return pl.pallas_call(
        _a_body,
        grid=(N_TOK // TILE + 1,),
        in_specs=[
            pl.BlockSpec((TILE, D), xmap),
            pl.BlockSpec((D, D), full2),
            pl.BlockSpec((1, D), full2),
            pl.BlockSpec((D, D), full2),
            pl.BlockSpec((1, D), full2),
            pl.BlockSpec((D, D), full2),
            pl.BlockSpec((1, D), full2),
            pl.BlockSpec((GLANES, D), full2),
            pl.BlockSpec((1, GLANES), full2),
        ],
        out_specs=[
            pl.BlockSpec((TILE, D), xmap),
            pl.BlockSpec((NW, 2, 64), lambda i: (0, 0, 0)),
            pl.BlockSpec((NW, 2, 64), lambda i: (0, 0, 0)),
            pl.BlockSpec((TILE, 16), xmap),
            pl.BlockSpec((TILE, 16), xmap),
            pl.BlockSpec((1, 128), full2),
        ],
        out_shape=[
            jax.ShapeDtypeStruct((N_TOK, D), jnp.float32),
            jax.ShapeDtypeStruct((NW, 2, 64), jnp.int32),
            jax.ShapeDtypeStruct((NW, 2, 64), jnp.int32),
            jax.ShapeDtypeStruct((N_TOK, 16), jnp.float32),
            jax.ShapeDtypeStruct((N_TOK, 16), jnp.float32),
            jax.ShapeDtypeStruct((1, 128), jnp.int32),
        ],
        scratch_shapes=[
            pltpu.VMEM((1, 128), jnp.float32),
            pltpu.VMEM((N_TOK, 16), jnp.float32),
        ],
    )(x, Wi, bi, Wv, bv, Wo, bo, Wg, bg)


def _dispatch(x2, pos0, pos1):
    mesh = plsc.VectorSubcoreMesh(core_axis_name="c", subcore_axis_name="s")

    @functools.partial(
        pl.kernel, mesh=mesh,
        out_type=jax.ShapeDtypeStruct((NDISP, D), jnp.float32),
        scratch_types=[
            pltpu.VMEM((CHUNK, D), jnp.float32),
            pltpu.VMEM((2, CHUNK), jnp.int32),
            pltpu.VMEM((2, CHUNK), jnp.int32),
            pltpu.SemaphoreType.DMA,
        ],
    )
    def k(x2_hbm, p0_hbm, p1_hbm, disp_hbm, rows_v, p0_v, p1_v, sem):
        wid = lax.axis_index("s") * 2 + lax.axis_index("c")
        pltpu.sync_copy(p0_hbm.at[wid], p0_v)
        pltpu.sync_copy(p1_hbm.at[wid], p1_v)
        for q in range(TPW // CHUNK):
            pltpu.sync_copy(x2_hbm.at[pl.ds(wid * TPW + q * CHUNK, CHUNK)],
                            rows_v)
            pltpu.async_copy(rows_v, disp_hbm.at[p0_v.at[q]], sem).wait()
            pltpu.async_copy(rows_v, disp_hbm.at[p1_v.at[q]], sem).wait()

    return k(x2, pos0, pos1)


def _ffn(disp, te_arr, W1, b1r, W2, b2r):
    grid_spec = pltpu.PrefetchScalarGridSpec(
        num_scalar_prefetch=1,
        grid=(NFT,),
        in_specs=[
            pl.BlockSpec((FTILE, D), lambda i, te: (i, 0)),
            pl.BlockSpec((1, DH, D), lambda i, te: (te[i], 0, 0)),
            pl.BlockSpec((1, 1, DH), lambda i, te: (te[i], 0, 0)),
            pl.BlockSpec((1, D, DH), lambda i, te: (te[i], 0, 0)),
            pl.BlockSpec((1, 1, D), lambda i, te: (te[i], 0, 0)),
        ],
        out_specs=pl.BlockSpec((FTILE, D), lambda i, te: (i, 0)),
    )

    def body(te_ref, d_ref, W1_ref, b1_ref, W2_ref, b2_ref, y_ref):
        h = jnp.maximum(_dg(d_ref[...], W1_ref[0]) + b1_ref[0], 0.0)
        y_ref[...] = _dg(h, W2_ref[0]) + b2_ref[0]

    return pl.pallas_call(
        body, grid_spec=grid_spec,
        out_shape=jax.ShapeDtypeStruct((NDISP, D), jnp.float32),
    )(te_arr, disp, W1, b1r, W2, b2r)


def _combine(y, pos0, pos1, w0, w1):
    mesh = plsc.VectorSubcoreMesh(core_axis_name="c", subcore_axis_name="s")

    @functools.partial(
        pl.kernel, mesh=mesh,
        out_type=jax.ShapeDtypeStruct((N_TOK, D), jnp.float32),
        scratch_types=[
            pltpu.VMEM((2, CHUNK), jnp.int32),
            pltpu.VMEM((2, CHUNK), jnp.int32),
            pltpu.VMEM((CHUNK, 16), jnp.float32),
            pltpu.VMEM((CHUNK, 16), jnp.float32),
            pltpu.VMEM((CHUNK, D), jnp.float32),
            pltpu.VMEM((CHUNK, D), jnp.float32),
            pltpu.SemaphoreType.DMA,
            pltpu.SemaphoreType.DMA,
        ],
    )
    def k(y_hbm, p0_hbm, p1_hbm, w0_hbm, w1_hbm, out_hbm,
          p0_v, p1_v, w0_v, w1_v, buf0, buf1, sem0, sem1):
        wid = lax.axis_index("s") * 2 + lax.axis_index("c")
        pltpu.sync_copy(p0_hbm.at[wid], p0_v)
        pltpu.sync_copy(p1_hbm.at[wid], p1_v)
        for half in range(TPW // CHUNK):
            base_tok = wid * TPW + half * CHUNK
            pltpu.sync_copy(w0_hbm.at[pl.ds(base_tok, CHUNK)], w0_v)
            pltpu.sync_copy(w1_hbm.at[pl.ds(base_tok, CHUNK)], w1_v)
            cp0 = pltpu.async_copy(y_hbm.at[p0_v.at[half]], buf0, sem0)
            cp1 = pltpu.async_copy(y_hbm.at[p1_v.at[half]], buf1, sem1)
            cp0.wait()
            cp1.wait()

            def tok(t, carry):
                w0s = w0_v[t, :]
                w1s = w1_v[t, :]

                def vv(vi, c2):
                    a = buf0[t, pl.ds(vi * 16, 16)]
                    b = buf1[t, pl.ds(vi * 16, 16)]
                    buf0[t, pl.ds(vi * 16, 16)] = a * w0s + b * w1s
                    return c2

                lax.fori_loop(0, D // 16, vv, 0)
                return carry

            lax.fori_loop(0, CHUNK, tok, 0)
            pltpu.sync_copy(
                buf0, out_hbm.at[pl.ds(wid * TPW + half * CHUNK, CHUNK)])

    return k(y, pos0, pos1, w0, w1)


def kernel(x, W_in, b_in, in_proj_w, in_proj_b, out_proj_w, out_proj_b,
           W_g, b_g, W1, b1, W2, b2):
    Wv = in_proj_w[2 * D:]
    bv = in_proj_b[2 * D:].reshape(1, D)
    Wg_pad = jnp.zeros((GLANES, D), jnp.float32).at[:E].set(W_g)
    bg_pad = jnp.full((1, GLANES), NEG, jnp.float32).at[0, :E].set(b_g)
    b1r = b1.reshape(E, 1, DH)
    b2r = b2.reshape(E, 1, D)

    x2, pos0, pos1, w0, w1, te = _route(
        x, W_in, b_in.reshape(1, D), Wv, bv, out_proj_w,
        out_proj_b.reshape(1, D), Wg_pad, bg_pad)
    te_arr = te[0, :NFT]
    disp = _dispatch(x2, pos0, pos1)
    y = _ffn(disp, te_arr, W1, b1r, W2, b2r)
    out = _combine(y, pos0, pos1, w0, w1)
    return out


# speedup vs baseline: 1.1690x; 1.1690x over previous
import functools

import jax
import jax.numpy as jnp
from jax import lax
from jax.experimental import pallas as pl
from jax.experimental.pallas import tpu as pltpu
from jax.experimental.pallas import tpu_sc as plsc

N_TOK = 4096
D = 768
E = 8
DH = 256
GLANES = 128
TILE = 512
FTILE = 256
NDISP = 2 * N_TOK + E * FTILE
NFT = NDISP // FTILE
NW = 32
TPW = N_TOK // NW
CHUNK = 64
NEG = -1e30


def _dg(a, b):
    return lax.dot_general(a, b, (((1,), (1,)), ((), ())),
                           preferred_element_type=jnp.float32)


def _top2(logits):
    iota = lax.broadcasted_iota(jnp.int32, logits.shape, 1)
    m1 = jnp.max(logits, axis=1, keepdims=True)
    i1 = jnp.min(jnp.where(logits == m1, iota, GLANES), axis=1, keepdims=True)
    lm = jnp.where(iota == i1, NEG, logits)
    m2 = jnp.max(lm, axis=1, keepdims=True)
    i2 = jnp.min(jnp.where(lm == m2, iota, GLANES), axis=1, keepdims=True)
    w1 = 1.0 / (1.0 + jnp.exp(m2 - m1))
    w2 = 1.0 - w1
    return i1, i2, w1, w2


def _a_body(x_ref, Wi_ref, bi_ref, Wv_ref, bv_ref, Wo_ref, bo_ref, Wg_ref,
            bg_ref, x2_ref, pos0_ref, pos1_ref, w0_ref, w1_ref, te_ref,
            carry_ref, meta_ref):
    i = pl.program_id(0)
    f32 = jnp.float32

    @pl.when(i == 0)
    def _init():
        carry_ref[...] = jnp.zeros((1, 128), f32)

    @pl.when(i < N_TOK // TILE)
    def _proj():
        x1 = _dg(x_ref[...], Wi_ref[...]) + bi_ref[...]
        v = _dg(x1, Wv_ref[...]) + bv_ref[...]
        x2 = _dg(v, Wo_ref[...]) + bo_ref[...]
        logits = _dg(x2, Wg_ref[...]) + bg_ref[...]
        i1, i2, wa, wb = _top2(logits)
        x2_ref[...] = x2
        w0_ref[...] = jnp.broadcast_to(wa, (TILE, 16))
        w1_ref[...] = jnp.broadcast_to(wb, (TILE, 16))
        ioE = lax.broadcasted_iota(jnp.int32, (TILE, E), 1)
        m = (ioE == i1).astype(f32) + (ioE == i2).astype(f32)
        r = lax.broadcasted_iota(jnp.int32, (TILE, TILE), 0)
        c = lax.broadcasted_iota(jnp.int32, (TILE, TILE), 1)
        tri = (c < r).astype(f32)
        pref = lax.dot_general(tri, m, (((1,), (0,)), ((), ())),
                               preferred_element_type=f32)
        mp = pref + carry_ref[0:1, 0:E]
        meta_ref[pl.ds(i * TILE, TILE), 0:E] = mp
        meta_ref[pl.ds(i * TILE, TILE), E:E + 1] = i1.astype(f32)
        meta_ref[pl.ds(i * TILE, TILE), E + 1:E + 2] = i2.astype(f32)
        carry_ref[0:1, 0:E] = carry_ref[0:1, 0:E] + jnp.sum(m, axis=0,
                                                            keepdims=True)

    @pl.when(i == N_TOK // TILE)
    def _plan():
        counts = carry_ref[0:1, 0:E].astype(jnp.int32)
        pc = ((counts + (FTILE - 1)) // FTILE) * FTILE
        er = lax.broadcasted_iota(jnp.int32, (E, E), 0)
        ec = lax.broadcasted_iota(jnp.int32, (E, E), 1)
        ut = (er < ec).astype(f32)
        seg = lax.dot_general(pc.astype(f32), ut, (((1,), (0,)), ((), ())),
                              preferred_element_type=f32)
        cumt = (seg.astype(jnp.int32) + pc) // FTILE
        tio = lax.broadcasted_iota(jnp.int32, (E, 128), 1)
        te = jnp.sum((tio >= cumt.reshape(E, 1)).astype(jnp.int32), axis=0,
                     keepdims=True)
        te_ref[...] = jnp.minimum(te, E - 1)
        for tt in range(N_TOK // TILE):
            blk = meta_ref[pl.ds(tt * TILE, TILE), :]
            mp = blk[:, 0:E] + seg
            i1f = blk[:, E:E + 1]
            i2f = blk[:, E + 1:E + 2]
            ioEf = lax.broadcasted_iota(jnp.int32, (TILE, E), 1).astype(f32)
            p0 = jnp.sum(jnp.where(ioEf == i1f, mp, 0.0), axis=1, keepdims=True)
            p1 = jnp.sum(jnp.where(ioEf == i2f, mp, 0.0), axis=1, keepdims=True)
            rpt = TILE // 128
            pos0_ref[pl.ds(tt * rpt, rpt), :, :] = p0.reshape(rpt, 2, 64).astype(jnp.int32)
            pos1_ref[pl.ds(tt * rpt, rpt), :, :] = p1.reshape(rpt, 2, 64).astype(jnp.int32)


def _route(x, Wi, bi, Wv, bv, Wo, bo, Wg, bg):
    full2 = lambda i: (0, 0)
    xmap = lambda i: (jnp.minimum(i, N_TOK // TILE - 1), 0)
    rpt = TILE // 128
    return pl.pallas_call(
        _a_body,
        grid=(N_TOK // TILE + 1,),
        in_specs=[
            pl.BlockSpec((TILE, D), xmap),
            pl.BlockSpec((D, D), full2),
            pl.BlockSpec((1, D), full2),
            pl.BlockSpec((D, D), full2),
            pl.BlockSpec((1, D), full2),
            pl.BlockSpec((D, D), full2),
            pl.BlockSpec((1, D), full2),
            pl.BlockSpec((GLANES, D), full2),
            pl.BlockSpec((1, GLANES), full2),
        ],
        out_specs=[
            pl.BlockSpec((TILE, D), xmap),
            pl.BlockSpec((NW, 2, 64), lambda i: (0, 0, 0)),
            pl.BlockSpec((NW, 2, 64), lambda i: (0, 0, 0)),
            pl.BlockSpec((TILE, 16), xmap),
            pl.BlockSpec((TILE, 16), xmap),
            pl.BlockSpec((1, 128), full2),
        ],
        out_shape=[
            jax.ShapeDtypeStruct((N_TOK, D), jnp.float32),
            jax.ShapeDtypeStruct((NW, 2, 64), jnp.int32),
            jax.ShapeDtypeStruct((NW, 2, 64), jnp.int32),
            jax.ShapeDtypeStruct((N_TOK, 16), jnp.float32),
            jax.ShapeDtypeStruct((N_TOK, 16), jnp.float32),
            jax.ShapeDtypeStruct((1, 128), jnp.int32),
        ],
        scratch_shapes=[
            pltpu.VMEM((1, 128), jnp.float32),
            pltpu.VMEM((N_TOK, 16), jnp.float32),
        ],
    )(x, Wi, bi, Wv, bv, Wo, bo, Wg, bg)


def _dispatch(x2, pos0, pos1):
    mesh = plsc.VectorSubcoreMesh(core_axis_name="c", subcore_axis_name="s")

    @functools.partial(
        pl.kernel, mesh=mesh,
        out_type=jax.ShapeDtypeStruct((NDISP, D), jnp.float32),
        scratch_types=[
            pltpu.VMEM((CHUNK, D), jnp.float32),
            pltpu.VMEM((CHUNK, D), jnp.float32),
            pltpu.VMEM((2, CHUNK), jnp.int32),
            pltpu.VMEM((2, CHUNK), jnp.int32),
            pltpu.SemaphoreType.DMA,
        ],
    )
    def k(x2_hbm, p0_hbm, p1_hbm, disp_hbm, rows_a, rows_b, p0_v, p1_v, sem):
        wid = lax.axis_index("s") * 2 + lax.axis_index("c")
        pltpu.sync_copy(p0_hbm.at[wid], p0_v)
        pltpu.sync_copy(p1_hbm.at[wid], p1_v)
        bufs = (rows_a, rows_b)
        pend = []
        for q in range(TPW // CHUNK):
            b = bufs[q % 2]
            pltpu.sync_copy(x2_hbm.at[pl.ds(wid * TPW + q * CHUNK, CHUNK)], b)
            for cp in pend:
                cp.wait()
            pend = [pltpu.async_copy(b, disp_hbm.at[p0_v.at[q]], sem),
                    pltpu.async_copy(b, disp_hbm.at[p1_v.at[q]], sem)]
        for cp in pend:
            cp.wait()

    return k(x2, pos0, pos1)


def _ffn(disp, te_arr, W1, b1r, W2, b2r):
    grid_spec = pltpu.PrefetchScalarGridSpec(
        num_scalar_prefetch=1,
        grid=(NFT,),
        in_specs=[
            pl.BlockSpec((FTILE, D), lambda i, te: (i, 0)),
            pl.BlockSpec((1, DH, D), lambda i, te: (te[i], 0, 0)),
            pl.BlockSpec((1, 1, DH), lambda i, te: (te[i], 0, 0)),
            pl.BlockSpec((1, D, DH), lambda i, te: (te[i], 0, 0)),
            pl.BlockSpec((1, 1, D), lambda i, te: (te[i], 0, 0)),
        ],
        out_specs=pl.BlockSpec((FTILE, D), lambda i, te: (i, 0)),
    )

    def body(te_ref, d_ref, W1_ref, b1_ref, W2_ref, b2_ref, y_ref):
        h = jnp.maximum(_dg(d_ref[...], W1_ref[0]) + b1_ref[0], 0.0)
        y_ref[...] = _dg(h, W2_ref[0]) + b2_ref[0]

    return pl.pallas_call(
        body, grid_spec=grid_spec,
        out_shape=jax.ShapeDtypeStruct((NDISP, D), jnp.float32),
    )(te_arr, disp, W1, b1r, W2, b2r)


def _combine(y, pos0, pos1, w0, w1):
    mesh = plsc.VectorSubcoreMesh(core_axis_name="c", subcore_axis_name="s")

    @functools.partial(
        pl.kernel, mesh=mesh,
        out_type=jax.ShapeDtypeStruct((N_TOK, D), jnp.float32),
        scratch_types=[
            pltpu.VMEM((2, CHUNK), jnp.int32),
            pltpu.VMEM((2, CHUNK), jnp.int32),
            pltpu.VMEM((CHUNK, 16), jnp.float32),
            pltpu.VMEM((CHUNK, 16), jnp.float32),
            pltpu.VMEM((CHUNK, D), jnp.float32),
            pltpu.VMEM((CHUNK, D), jnp.float32),
            pltpu.SemaphoreType.DMA,
            pltpu.SemaphoreType.DMA,
        ],
    )
    def k(y_hbm, p0_hbm, p1_hbm, w0_hbm, w1_hbm, out_hbm,
          p0_v, p1_v, w0_v, w1_v, buf0, buf1, sem0, sem1):
        wid = lax.axis_index("s") * 2 + lax.axis_index("c")
        pltpu.sync_copy(p0_hbm.at[wid], p0_v)
        pltpu.sync_copy(p1_hbm.at[wid], p1_v)
        for half in range(TPW // CHUNK):
            base_tok = wid * TPW + half * CHUNK
            pltpu.sync_copy(w0_hbm.at[pl.ds(base_tok, CHUNK)], w0_v)
            pltpu.sync_copy(w1_hbm.at[pl.ds(base_tok, CHUNK)], w1_v)
            cp0 = pltpu.async_copy(y_hbm.at[p0_v.at[half]], buf0, sem0)
            cp1 = pltpu.async_copy(y_hbm.at[p1_v.at[half]], buf1, sem1)
            cp0.wait()
            cp1.wait()

            def tok(t, carry):
                w0s = w0_v[t, :]
                w1s = w1_v[t, :]
                for vi in range(D // 16):
                    a = buf0[t, pl.ds(vi * 16, 16)]
                    b = buf1[t, pl.ds(vi * 16, 16)]
                    buf0[t, pl.ds(vi * 16, 16)] = a * w0s + b * w1s
                return carry

            lax.fori_loop(0, CHUNK, tok, 0)
            pltpu.sync_copy(
                buf0, out_hbm.at[pl.ds(wid * TPW + half * CHUNK, CHUNK)])

    return k(y, pos0, pos1, w0, w1)


def kernel(x, W_in, b_in, in_proj_w, in_proj_b, out_proj_w, out_proj_b,
           W_g, b_g, W1, b1, W2, b2):
    Wv = in_proj_w[2 * D:]
    bv = in_proj_b[2 * D:].reshape(1, D)
    Wg_pad = jnp.zeros((GLANES, D), jnp.float32).at[:E].set(W_g)
    bg_pad = jnp.full((1, GLANES), NEG, jnp.float32).at[0, :E].set(b_g)
    b1r = b1.reshape(E, 1, DH)
    b2r = b2.reshape(E, 1, D)

    x2, pos0, pos1, w0, w1, te = _route(
        x, W_in, b_in.reshape(1, D), Wv, bv, out_proj_w,
        out_proj_b.reshape(1, D), Wg_pad, bg_pad)
    te_arr = te[0, :NFT]
    disp = _dispatch(x2, pos0, pos1)
    y = _ffn(disp, te_arr, W1, b1r, W2, b2r)
    out = _combine(y, pos0, pos1, w0, w1)
    return out
